# baseline (speedup 1.0000x reference)
import functools

import jax
import jax.numpy as jnp
from jax.experimental import pallas as pl
from jax.experimental.pallas import tpu as pltpu

_K = 20
_EPS = 1e-5
_NB = 256


def _edge_kernel(xb_ref, xtf_ref, w_ref, outmax_ref, outmin_ref, stats_ref,
                 blk_ref):
    b = pl.program_id(0)
    i = pl.program_id(1)
    xb = xb_ref[0]
    xtf = xtf_ref[0]
    n = xtf.shape[1]

    s = jax.lax.dot_general(xb, xtf, (((1,), (0,)), ((), ())),
                            preferred_element_type=jnp.float32)
    inner_neg = -2.0 * s
    xxb = (xb[:, 0:1] * xb[:, 0:1] + xb[:, 1:2] * xb[:, 1:2]
           + xb[:, 2:3] * xb[:, 2:3])
    xxf = jnp.sum(xtf * xtf, axis=0, keepdims=True)
    d = (-xxb - inner_neg) - xxf

    wd = w_ref[0:3, :]
    wq = w_ref[3:6, :]
    p = jax.lax.dot_general(xtf, wd, (((0,), (0,)), ((), ())),
                            preferred_element_type=jnp.float32)
    q = jax.lax.dot_general(xb, wq, (((1,), (0,)), ((), ())),
                            preferred_element_type=jnp.float32)

    paug = jnp.concatenate([p, jnp.ones((n, 1), jnp.float32)], axis=1)

    neg = jnp.float32(-jnp.inf)
    nb = d.shape[0]

    def finalize(m_max, m_min, m_sum, m_sum2):
        outmax_ref[0] = m_max + q
        outmin_ref[0] = m_min + q
        s1 = jnp.sum(m_sum, axis=0, keepdims=True)
        s2 = jnp.sum(m_sum2, axis=0, keepdims=True)
        s3 = jnp.sum(q * m_sum, axis=0, keepdims=True)
        s4 = jnp.sum(q, axis=0, keepdims=True)
        s5 = jnp.sum(q * q, axis=0, keepdims=True)
        z = jnp.zeros_like(s1)
        blk_ref[...] = jnp.concatenate([s1, s2, s3, s4, s5, z, z, z], axis=0)

    df = d
    m_max = jnp.full((nb, 64), -jnp.inf, jnp.float32)
    m_min = jnp.full((nb, 64), jnp.inf, jnp.float32)
    m_sum = jnp.zeros((nb, 64), jnp.float32)
    m_sum2 = jnp.zeros((nb, 64), jnp.float32)
    m_cnt = jnp.zeros((nb, 1), jnp.float32)
    for _ in range(_K):
        v = jnp.max(df, axis=1, keepdims=True)
        sel = df == v
        df = jnp.where(sel, neg, df)
        onehot = jnp.where(sel, 1.0, 0.0).astype(jnp.float32)
        selp = jax.lax.dot_general(onehot, paug, (((1,), (0,)), ((), ())),
                                   preferred_element_type=jnp.float32)
        m_max = jnp.maximum(m_max, selp[:, :64])
        m_min = jnp.minimum(m_min, selp[:, :64])
        m_sum = m_sum + selp[:, :64]
        m_sum2 = m_sum2 + selp[:, :64] * selp[:, :64]
        m_cnt = m_cnt + selp[:, 64:65]
    finalize(m_max, m_min, m_sum, m_sum2)

    @pl.when(jnp.max(m_cnt) > _K + 0.5)
    def _():
        iota = jax.lax.broadcasted_iota(
            jnp.int32, d.shape, 1).astype(jnp.float32)
        nf = jnp.float32(n)
        de = d
        e_max = jnp.full((nb, 64), -jnp.inf, jnp.float32)
        e_min = jnp.full((nb, 64), jnp.inf, jnp.float32)
        e_sum = jnp.zeros((nb, 64), jnp.float32)
        e_sum2 = jnp.zeros((nb, 64), jnp.float32)
        for _ in range(_K):
            v = jnp.max(de, axis=1, keepdims=True)
            cand = jnp.where(de == v, iota, nf)
            idx = jnp.min(cand, axis=1, keepdims=True)
            sel = cand == idx
            de = jnp.where(sel, neg, de)
            onehot = jnp.where(sel, 1.0, 0.0).astype(jnp.float32)
            selp = jax.lax.dot_general(onehot, p, (((1,), (0,)), ((), ())),
                                       preferred_element_type=jnp.float32)
            e_max = jnp.maximum(e_max, selp)
            e_min = jnp.minimum(e_min, selp)
            e_sum = e_sum + selp
            e_sum2 = e_sum2 + selp * selp
        finalize(e_max, e_min, e_sum, e_sum2)

    @pl.when((b == 0) & (i == 0))
    def _():
        stats_ref[...] = jnp.zeros_like(stats_ref)

    stats_ref[...] += blk_ref[...]


def _mlp_kernel(hmax_ref, hmin_ref, stats_ref, gc_ref, bc_ref,
                w1_ref, b1_ref, g1_ref, t1_ref,
                w2_ref, b2_ref, g2_ref, t2_ref,
                w3_ref, b3_ref, g3_ref, t3_ref,
                w4_ref, b4_ref, w5_ref, b5_ref, o_ref, *, nbatch, npts):
    cnt = jnp.float32(nbatch * npts * _K)
    st = stats_ref[...]
    s1 = st[0:1, :]
    s2 = st[1:2, :]
    s3 = st[2:3, :]
    s4 = st[3:4, :]
    s5 = st[4:5, :]
    mean = (s1 + _K * s4) / cnt
    ex2 = (s2 + 2.0 * s3 + _K * s5) / cnt
    var = ex2 - mean * mean
    inv = gc_ref[...] / jnp.sqrt(var + _EPS)

    m = jnp.where(inv >= 0, hmax_ref[...], hmin_ref[...])
    h = (m - mean) * inv + bc_ref[...]
    h0 = jnp.where(h >= 0, h, 0.2 * h)

    def dense_bn_relu(a, w_ref_, b_ref_, g_ref_, t_ref_):
        a = jax.lax.dot_general(a, w_ref_[...], (((1,), (0,)), ((), ())),
                                preferred_element_type=jnp.float32)
        a = a + b_ref_[...]
        mu = jnp.sum(a, axis=0, keepdims=True) / a.shape[0]
        c = a - mu
        va = jnp.sum(c * c, axis=0, keepdims=True) / a.shape[0]
        return jnp.maximum(c / jnp.sqrt(va + _EPS) * g_ref_[...] + t_ref_[...],
                           0.0)

    h1 = dense_bn_relu(h0, w1_ref, b1_ref, g1_ref, t1_ref)
    h2 = dense_bn_relu(h1, w2_ref, b2_ref, g2_ref, t2_ref)
    h3 = dense_bn_relu(h2, w3_ref, b3_ref, g3_ref, t3_ref)

    hb = jnp.max(h3.reshape(nbatch, npts, h3.shape[1]), axis=1)
    h4 = jax.lax.dot_general(hb, w4_ref[...], (((1,), (0,)), ((), ())),
                             preferred_element_type=jnp.float32)
    h4 = jnp.maximum(h4 + b4_ref[...], 0.0)
    out = jax.lax.dot_general(h4, w5_ref[...], (((1,), (0,)), ((), ())),
                              preferred_element_type=jnp.float32)
    o_ref[...] = out + b5_ref[...]


def kernel(x, W_conv, g_conv, b_conv, W1, b1, g1, bt1, W2, b2, g2, bt2,
           W3, b3, g3, bt3, W4, b4, W5, b5):
    B, N, _ = x.shape
    xt = jnp.transpose(x, (0, 2, 1))
    wd = jnp.transpose(W_conv[:, :3])
    wq = jnp.transpose(W_conv[:, 3:] - W_conv[:, :3])
    w = jnp.concatenate([wd, wq, jnp.zeros((2, 64), jnp.float32)], axis=0)

    nblk = N // _NB
    hmax, hmin, stats = pl.pallas_call(
        _edge_kernel,
        grid=(B, nblk),
        in_specs=[
            pl.BlockSpec((1, _NB, 3), lambda b, i: (b, i, 0)),
            pl.BlockSpec((1, 3, N), lambda b, i: (b, 0, 0)),
            pl.BlockSpec((8, 64), lambda b, i: (0, 0)),
        ],
        out_specs=[
            pl.BlockSpec((1, _NB, 64), lambda b, i: (b, i, 0)),
            pl.BlockSpec((1, _NB, 64), lambda b, i: (b, i, 0)),
            pl.BlockSpec((8, 64), lambda b, i: (0, 0)),
        ],
        out_shape=[
            jax.ShapeDtypeStruct((B, N, 64), jnp.float32),
            jax.ShapeDtypeStruct((B, N, 64), jnp.float32),
            jax.ShapeDtypeStruct((8, 64), jnp.float32),
        ],
        scratch_shapes=[pltpu.VMEM((8, 64), jnp.float32)],
    )(x, xt, w)

    row = lambda v: v.reshape(1, -1)
    out = pl.pallas_call(
        functools.partial(_mlp_kernel, nbatch=B, npts=N),
        out_shape=jax.ShapeDtypeStruct((B, 2 * 128), jnp.float32),
    )(hmax.reshape(B * N, 64), hmin.reshape(B * N, 64), stats,
      row(g_conv), row(b_conv),
      W1, row(b1), row(g1), row(bt1),
      W2, row(b2), row(g2), row(bt2),
      W3, row(b3), row(g3), row(bt3),
      W4, row(b4), W5, row(b5))
    return out
